# baseline (speedup 1.0000x reference)
import functools

import jax
import jax.numpy as jnp
from jax import lax
from jax.experimental import pallas as pl
from jax.experimental.pallas import tpu as pltpu
from jax.experimental.pallas import tpu_sc as plsc

N_NODES = 10000
N_EDGES = 320000
D_IN = 512
D_H1 = 256
D_H2 = 16

NC = 2
NS = 16
NW = NC * NS
LANES = 16

EB = 125
EROWS = N_EDGES // EB

_sc_mesh = plsc.VectorSubcoreMesh(core_axis_name="c", subcore_axis_name="s")
_sc_params = pltpu.CompilerParams(needs_layout_passes=False,
                                  use_tc_tiling_on_sc=False)


def _deg_body(src_hbm, dst_hbm, out_s_hbm, out_d_hbm, src_v, dst_v,
              acc_s, acc_d, sem):
    c = lax.axis_index("c")
    s = lax.axis_index("s")
    wid = c * NS + s
    epw = N_EDGES // NW

    cp1 = pltpu.async_copy(src_hbm.at[pl.ds(wid * epw, epw)], src_v, sem)
    cp2 = pltpu.async_copy(dst_hbm.at[pl.ds(wid * epw, epw)], dst_v, sem)

    zeros = jnp.zeros((LANES,), jnp.float32)

    def zero_body(i):
        acc_s[pl.ds(i * LANES, LANES)] = zeros
        acc_d[pl.ds(i * LANES, LANES)] = zeros

    pl.loop(0, N_NODES // LANES, unroll=8)(zero_body)
    cp1.wait()
    cp2.wait()

    ones = jnp.ones((LANES,), jnp.float32)

    def body(i):
        plsc.addupdate_scatter(acc_s, [src_v[pl.ds(i * LANES, LANES)]], ones)
        plsc.addupdate_scatter(acc_d, [dst_v[pl.ds(i * LANES, LANES)]], ones)

    pl.loop(0, epw // LANES, unroll=8)(body)

    pltpu.sync_copy(acc_s, out_s_hbm.at[pl.ds(wid * N_NODES, N_NODES)])
    pltpu.sync_copy(acc_d, out_d_hbm.at[pl.ds(wid * N_NODES, N_NODES)])


_deg_kernel = functools.partial(
    pl.kernel,
    out_type=(
        jax.ShapeDtypeStruct((NW * N_NODES,), jnp.float32),
        jax.ShapeDtypeStruct((NW * N_NODES,), jnp.float32),
    ),
    mesh=_sc_mesh,
    scratch_types=[
        pltpu.VMEM((N_EDGES // NW,), jnp.int32),
        pltpu.VMEM((N_EDGES // NW,), jnp.int32),
        pltpu.VMEM((N_NODES,), jnp.float32),
        pltpu.VMEM((N_NODES,), jnp.float32),
        pltpu.SemaphoreType.DMA,
    ],
    compiler_params=_sc_params,
)(_deg_body)


def _scales_body(ps_ref, pd_ref, so_ref, si_ref):
    d_out = jnp.sum(ps_ref[...], axis=0)
    d_in = jnp.sum(pd_ref[...], axis=0)
    so_ref[...] = lax.rsqrt(jnp.clip(d_out, 1.0, None))[:, None]
    si_ref[...] = lax.rsqrt(jnp.clip(d_in, 1.0, None))[:, None]


def _scales(part_s, part_d):
    return pl.pallas_call(
        _scales_body,
        out_shape=(
            jax.ShapeDtypeStruct((N_NODES, 1), jnp.float32),
            jax.ShapeDtypeStruct((N_NODES, 1), jnp.float32),
        ),
    )(part_s, part_d)


def _mm1_body(x_ref, w_ref, so_ref, out_ref):
    acc = jnp.dot(x_ref[...], w_ref[...], preferred_element_type=jnp.float32)
    acc = acc * so_ref[...]
    out_ref[0] = acc[:, :128]
    out_ref[1] = acc[:, 128:]


def _mm1(features, W1, s_out):
    blk = 1000
    grid = N_NODES // blk
    return pl.pallas_call(
        _mm1_body,
        grid=(grid,),
        in_specs=[
            pl.BlockSpec((blk, D_IN), lambda i: (i, 0)),
            pl.BlockSpec((D_IN, D_H1), lambda i: (0, 0)),
            pl.BlockSpec((blk, 1), lambda i: (i, 0)),
        ],
        out_specs=pl.BlockSpec((2, blk, 128), lambda i: (0, i, 0)),
        out_shape=jax.ShapeDtypeStruct((2, N_NODES, 128), jnp.float32),
    )(features, W1, s_out)


def _agg_pipelined(table, src_v, dst_v, rows, gsems, acc_sh, ch):
    nbuf = len(rows)
    for b in range(nbuf - 1):
        pltpu.async_copy(table.at[src_v.at[b]], rows[b], gsems[b])

    def group(jj):
        for b in range(nbuf):
            j = nbuf * jj + b
            nxt = j + nbuf - 1
            nb = (b + nbuf - 1) % nbuf

            @pl.when(nxt < ch)
            def _():
                pltpu.async_copy(table.at[src_v.at[nxt]], rows[nb], gsems[nb])

            pltpu.make_async_copy(table.at[src_v.at[j]],
                                  rows[b], gsems[b]).wait()
            pltpu.sync_copy(rows[b], acc_sh.at[dst_v.at[j]], add=True)

    pl.loop(0, ch // nbuf, unroll=2)(group)


def _par_copy(src, dst, s):
    @pl.when(s < 15)
    def _():
        pltpu.sync_copy(src.at[pl.ds(s * 640, 640)],
                        dst.at[pl.ds(s * 640, 640)])

    @pl.when(s == 15)
    def _():
        pltpu.sync_copy(src.at[pl.ds(9600, 400)], dst.at[pl.ds(9600, 400)])


def _agg1_body(x1_hbm, src_hbm, dst_hbm, zero_hbm, out_hbm,
               src_v, dst_v, rows0, rows1, acc_sh, sem, gsem0, gsem1):
    c = lax.axis_index("c")
    s = lax.axis_index("s")
    rpw = EROWS // NS
    ch = 40

    _par_copy(zero_hbm, acc_sh, s)
    plsc.subcore_barrier()

    def chunk(kc):
        cp1 = pltpu.async_copy(src_hbm.at[pl.ds(s * rpw + kc * ch, ch)],
                               src_v, sem)
        cp2 = pltpu.async_copy(dst_hbm.at[pl.ds(s * rpw + kc * ch, ch)],
                               dst_v, sem)
        cp1.wait()
        cp2.wait()
        _agg_pipelined(x1_hbm.at[c], src_v, dst_v, (rows0, rows1),
                       (gsem0, gsem1), acc_sh, ch)

    pl.loop(0, rpw // ch)(chunk)
    plsc.subcore_barrier()
    _par_copy(acc_sh, out_hbm.at[c], s)


_agg1_kernel = functools.partial(
    pl.kernel,
    out_type=jax.ShapeDtypeStruct((2, N_NODES, 128), jnp.float32),
    mesh=_sc_mesh,
    scratch_types=[
        pltpu.VMEM((40, EB), jnp.int32),
        pltpu.VMEM((40, EB), jnp.int32),
        pltpu.VMEM((EB, 128), jnp.float32),
        pltpu.VMEM((EB, 128), jnp.float32),
        pltpu.VMEM_SHARED((N_NODES, 128), jnp.float32),
        pltpu.SemaphoreType.DMA,
        pltpu.SemaphoreType.DMA,
        pltpu.SemaphoreType.DMA,
    ],
    compiler_params=_sc_params,
)(_agg1_body)


def _mm2_body(a_ref, si_ref, so_ref, b1_ref, w_ref, out_ref):
    si = si_ref[...]
    so = so_ref[...]
    h0 = jnp.maximum(a_ref[0] * si + b1_ref[0, :128][None, :], 0.0) * so
    h1 = jnp.maximum(a_ref[1] * si + b1_ref[0, 128:][None, :], 0.0) * so
    out_ref[...] = (
        jnp.dot(h0, w_ref[:128], preferred_element_type=jnp.float32)
        + jnp.dot(h1, w_ref[128:], preferred_element_type=jnp.float32)
    )


def _mm2(agg1, s_in, s_out, b1, W2):
    blk = 1000
    grid = N_NODES // blk
    return pl.pallas_call(
        _mm2_body,
        grid=(grid,),
        in_specs=[
            pl.BlockSpec((2, blk, 128), lambda i: (0, i, 0)),
            pl.BlockSpec((blk, 1), lambda i: (i, 0)),
            pl.BlockSpec((blk, 1), lambda i: (i, 0)),
            pl.BlockSpec((1, D_H1), lambda i: (0, 0)),
            pl.BlockSpec((D_H1, D_H2), lambda i: (0, 0)),
        ],
        out_specs=pl.BlockSpec((blk, D_H2), lambda i: (i, 0)),
        out_shape=jax.ShapeDtypeStruct((N_NODES, D_H2), jnp.float32),
    )(agg1, s_in, s_out, b1, W2)


def _agg2_body(x2_hbm, src_hbm, dst_hbm, zero_hbm, out_hbm,
               src_v, dst_v, r0, r1, r2, r3, acc_sh,
               sem, gs0, gs1, gs2, gs3):
    c = lax.axis_index("c")
    s = lax.axis_index("s")
    rpw = EROWS // NW
    row0 = (c * NS + s) * rpw

    cp1 = pltpu.async_copy(src_hbm.at[pl.ds(row0, rpw)], src_v, sem)
    cp2 = pltpu.async_copy(dst_hbm.at[pl.ds(row0, rpw)], dst_v, sem)
    _par_copy(zero_hbm, acc_sh, s)
    cp1.wait()
    cp2.wait()
    plsc.subcore_barrier()
    _agg_pipelined(x2_hbm, src_v, dst_v, (r0, r1, r2, r3),
                   (gs0, gs1, gs2, gs3), acc_sh, rpw)
    plsc.subcore_barrier()
    _par_copy(acc_sh, out_hbm.at[c], s)


_agg2_kernel = functools.partial(
    pl.kernel,
    out_type=jax.ShapeDtypeStruct((2, N_NODES, D_H2), jnp.float32),
    mesh=_sc_mesh,
    scratch_types=[
        pltpu.VMEM((EROWS // NW, EB), jnp.int32),
        pltpu.VMEM((EROWS // NW, EB), jnp.int32),
        pltpu.VMEM((EB, D_H2), jnp.float32),
        pltpu.VMEM((EB, D_H2), jnp.float32),
        pltpu.VMEM((EB, D_H2), jnp.float32),
        pltpu.VMEM((EB, D_H2), jnp.float32),
        pltpu.VMEM_SHARED((N_NODES, D_H2), jnp.float32),
        pltpu.SemaphoreType.DMA,
        pltpu.SemaphoreType.DMA,
        pltpu.SemaphoreType.DMA,
        pltpu.SemaphoreType.DMA,
        pltpu.SemaphoreType.DMA,
    ],
    compiler_params=_sc_params,
)(_agg2_body)


def _z_body(p_ref, si_ref, b2_ref, z_ref):
    z_ref[...] = (p_ref[0] + p_ref[1]) * si_ref[...] + b2_ref[0][None, :]


def _z_combine(partials, s_in, b2):
    return pl.pallas_call(
        _z_body,
        out_shape=jax.ShapeDtypeStruct((N_NODES, D_H2), jnp.float32),
    )(partials, s_in, b2)


def _dec_body(zi_ref, zj_ref, out_ref):
    prod = lax.dot_general(zi_ref[...], zj_ref[...],
                           (((1,), (1,)), ((), ())),
                           preferred_element_type=jnp.float32)
    out_ref[...] = jax.nn.sigmoid(prod)


def _decoder(z):
    blk = 400
    grid = N_NODES // blk
    return pl.pallas_call(
        _dec_body,
        grid=(grid,),
        in_specs=[
            pl.BlockSpec((blk, D_H2), lambda i: (i, 0)),
            pl.BlockSpec((N_NODES, D_H2), lambda i: (0, 0)),
        ],
        out_specs=pl.BlockSpec((blk, N_NODES), lambda i: (i, 0)),
        out_shape=jax.ShapeDtypeStruct((N_NODES, N_NODES), jnp.float32),
    )(z, z)


@jax.jit
def kernel(features, edge_index, W1, b1, W2, b2):
    src = edge_index[0]
    dst = edge_index[1]
    src2d = src.reshape(EROWS, EB)
    dst2d = dst.reshape(EROWS, EB)

    part_s, part_d = _deg_kernel(src, dst)
    s_out, s_in = _scales(part_s.reshape(NW, N_NODES), part_d.reshape(NW, N_NODES))

    x1 = _mm1(features, W1, s_out)
    zero1 = jnp.zeros((N_NODES, 128), jnp.float32)
    agg1 = _agg1_kernel(x1, src2d, dst2d, zero1)

    x2 = _mm2(agg1, s_in, s_out, b1.reshape(1, D_H1), W2)
    zero2 = jnp.zeros((N_NODES, D_H2), jnp.float32)
    p2 = _agg2_kernel(x2, src2d, dst2d, zero2)

    z = _z_combine(p2, s_in, b2.reshape(1, D_H2))
    return _decoder(z)


# speedup vs baseline: 1.0125x; 1.0125x over previous
import functools

import jax
import jax.numpy as jnp
from jax import lax
from jax.experimental import pallas as pl
from jax.experimental.pallas import tpu as pltpu
from jax.experimental.pallas import tpu_sc as plsc

N_NODES = 10000
N_EDGES = 320000
D_IN = 512
D_H1 = 256
D_H2 = 16

NC = 2
NS = 16
NW = NC * NS
LANES = 16

EB = 125
EROWS = N_EDGES // EB

_sc_mesh = plsc.VectorSubcoreMesh(core_axis_name="c", subcore_axis_name="s")
_sc_params = pltpu.CompilerParams(needs_layout_passes=False,
                                  use_tc_tiling_on_sc=False)


def _deg_body(src_hbm, dst_hbm, out_s_hbm, out_d_hbm, src_v, dst_v,
              acc_s, acc_d, sem):
    c = lax.axis_index("c")
    s = lax.axis_index("s")
    wid = c * NS + s
    epw = N_EDGES // NW

    cp1 = pltpu.async_copy(src_hbm.at[pl.ds(wid * epw, epw)], src_v, sem)
    cp2 = pltpu.async_copy(dst_hbm.at[pl.ds(wid * epw, epw)], dst_v, sem)

    zeros = jnp.zeros((LANES,), jnp.float32)

    def zero_body(i):
        acc_s[pl.ds(i * LANES, LANES)] = zeros
        acc_d[pl.ds(i * LANES, LANES)] = zeros

    pl.loop(0, N_NODES // LANES, unroll=8)(zero_body)
    cp1.wait()
    cp2.wait()

    ones = jnp.ones((LANES,), jnp.float32)

    def body(i):
        plsc.addupdate_scatter(acc_s, [src_v[pl.ds(i * LANES, LANES)]], ones)
        plsc.addupdate_scatter(acc_d, [dst_v[pl.ds(i * LANES, LANES)]], ones)

    pl.loop(0, epw // LANES, unroll=8)(body)

    pltpu.sync_copy(acc_s, out_s_hbm.at[pl.ds(wid * N_NODES, N_NODES)])
    pltpu.sync_copy(acc_d, out_d_hbm.at[pl.ds(wid * N_NODES, N_NODES)])


_deg_kernel = functools.partial(
    pl.kernel,
    out_type=(
        jax.ShapeDtypeStruct((NW * N_NODES,), jnp.float32),
        jax.ShapeDtypeStruct((NW * N_NODES,), jnp.float32),
    ),
    mesh=_sc_mesh,
    scratch_types=[
        pltpu.VMEM((N_EDGES // NW,), jnp.int32),
        pltpu.VMEM((N_EDGES // NW,), jnp.int32),
        pltpu.VMEM((N_NODES,), jnp.float32),
        pltpu.VMEM((N_NODES,), jnp.float32),
        pltpu.SemaphoreType.DMA,
    ],
    compiler_params=_sc_params,
)(_deg_body)


def _scales_body(ps_ref, pd_ref, so_ref, si_ref):
    d_out = jnp.sum(ps_ref[...], axis=0)
    d_in = jnp.sum(pd_ref[...], axis=0)
    so_ref[...] = lax.rsqrt(jnp.clip(d_out, 1.0, None))[:, None]
    si_ref[...] = lax.rsqrt(jnp.clip(d_in, 1.0, None))[:, None]


def _scales(part_s, part_d):
    return pl.pallas_call(
        _scales_body,
        out_shape=(
            jax.ShapeDtypeStruct((N_NODES, 1), jnp.float32),
            jax.ShapeDtypeStruct((N_NODES, 1), jnp.float32),
        ),
    )(part_s, part_d)


def _mm1_body(x_ref, w_ref, so_ref, out_ref):
    acc = jnp.dot(x_ref[...], w_ref[...], preferred_element_type=jnp.float32)
    acc = acc * so_ref[...]
    out_ref[0] = acc[:, :128]
    out_ref[1] = acc[:, 128:]


def _mm1(features, W1, s_out):
    blk = 1000
    grid = N_NODES // blk
    return pl.pallas_call(
        _mm1_body,
        grid=(grid,),
        in_specs=[
            pl.BlockSpec((blk, D_IN), lambda i: (i, 0)),
            pl.BlockSpec((D_IN, D_H1), lambda i: (0, 0)),
            pl.BlockSpec((blk, 1), lambda i: (i, 0)),
        ],
        out_specs=pl.BlockSpec((2, blk, 128), lambda i: (0, i, 0)),
        out_shape=jax.ShapeDtypeStruct((2, N_NODES, 128), jnp.float32),
    )(features, W1, s_out)


def _agg_pipelined(table, src_v, dst_v, rows, gsems, acc_sh, ch):
    nbuf = len(rows)
    for b in range(nbuf - 1):
        pltpu.async_copy(table.at[src_v.at[b]], rows[b], gsems[b])

    def group(jj):
        for b in range(nbuf):
            j = nbuf * jj + b
            nxt = j + nbuf - 1
            nb = (b + nbuf - 1) % nbuf

            @pl.when(nxt < ch)
            def _():
                pltpu.async_copy(table.at[src_v.at[nxt]], rows[nb], gsems[nb])

            pltpu.make_async_copy(table.at[src_v.at[j]],
                                  rows[b], gsems[b]).wait()
            pltpu.sync_copy(rows[b], acc_sh.at[dst_v.at[j]], add=True)

    pl.loop(0, ch // nbuf, unroll=2)(group)


def _par_copy(src, dst, s):
    @pl.when(s < 15)
    def _():
        pltpu.sync_copy(src.at[pl.ds(s * 640, 640)],
                        dst.at[pl.ds(s * 640, 640)])

    @pl.when(s == 15)
    def _():
        pltpu.sync_copy(src.at[pl.ds(9600, 400)], dst.at[pl.ds(9600, 400)])


def _agg1_body(x1_hbm, src_hbm, dst_hbm, zero_hbm, out_hbm,
               src_v, dst_v, rows0, rows1, acc_sh, sem, gsem0, gsem1):
    c = lax.axis_index("c")
    s = lax.axis_index("s")
    rpw = EROWS // NS
    ch = 40

    _par_copy(zero_hbm, acc_sh, s)
    plsc.subcore_barrier()

    def chunk(kc):
        cp1 = pltpu.async_copy(src_hbm.at[pl.ds(s * rpw + kc * ch, ch)],
                               src_v, sem)
        cp2 = pltpu.async_copy(dst_hbm.at[pl.ds(s * rpw + kc * ch, ch)],
                               dst_v, sem)
        cp1.wait()
        cp2.wait()
        _agg_pipelined(x1_hbm.at[c], src_v, dst_v, (rows0, rows1),
                       (gsem0, gsem1), acc_sh, ch)

    pl.loop(0, rpw // ch)(chunk)
    plsc.subcore_barrier()
    _par_copy(acc_sh, out_hbm.at[c], s)


_agg1_kernel = functools.partial(
    pl.kernel,
    out_type=jax.ShapeDtypeStruct((2, N_NODES, 128), jnp.float32),
    mesh=_sc_mesh,
    scratch_types=[
        pltpu.VMEM((40, EB), jnp.int32),
        pltpu.VMEM((40, EB), jnp.int32),
        pltpu.VMEM((EB, 128), jnp.float32),
        pltpu.VMEM((EB, 128), jnp.float32),
        pltpu.VMEM_SHARED((N_NODES, 128), jnp.float32),
        pltpu.SemaphoreType.DMA,
        pltpu.SemaphoreType.DMA,
        pltpu.SemaphoreType.DMA,
    ],
    compiler_params=_sc_params,
)(_agg1_body)


def _mm2_body(a_ref, si_ref, so_ref, b1_ref, w_ref, out_ref):
    si = si_ref[...]
    so = so_ref[...]
    h0 = jnp.maximum(a_ref[0] * si + b1_ref[0, :128][None, :], 0.0) * so
    h1 = jnp.maximum(a_ref[1] * si + b1_ref[0, 128:][None, :], 0.0) * so
    out_ref[...] = (
        jnp.dot(h0, w_ref[:128], preferred_element_type=jnp.float32)
        + jnp.dot(h1, w_ref[128:], preferred_element_type=jnp.float32)
    )


def _mm2(agg1, s_in, s_out, b1, W2):
    blk = 1000
    grid = N_NODES // blk
    return pl.pallas_call(
        _mm2_body,
        grid=(grid,),
        in_specs=[
            pl.BlockSpec((2, blk, 128), lambda i: (0, i, 0)),
            pl.BlockSpec((blk, 1), lambda i: (i, 0)),
            pl.BlockSpec((blk, 1), lambda i: (i, 0)),
            pl.BlockSpec((1, D_H1), lambda i: (0, 0)),
            pl.BlockSpec((D_H1, D_H2), lambda i: (0, 0)),
        ],
        out_specs=pl.BlockSpec((blk, D_H2), lambda i: (i, 0)),
        out_shape=jax.ShapeDtypeStruct((N_NODES, D_H2), jnp.float32),
    )(agg1, s_in, s_out, b1, W2)


def _agg2_body(x2_hbm, src_hbm, dst_hbm, zero_hbm, out_hbm,
               src_v, dst_v, r0, r1, r2, r3, acc_sh, x2_sh,
               sem, gs0, gs1, gs2, gs3):
    c = lax.axis_index("c")
    s = lax.axis_index("s")
    rpw = EROWS // NW
    row0 = (c * NS + s) * rpw

    cp1 = pltpu.async_copy(src_hbm.at[pl.ds(row0, rpw)], src_v, sem)
    cp2 = pltpu.async_copy(dst_hbm.at[pl.ds(row0, rpw)], dst_v, sem)
    _par_copy(zero_hbm, acc_sh, s)
    _par_copy(x2_hbm, x2_sh, s)
    cp1.wait()
    cp2.wait()
    plsc.subcore_barrier()
    _agg_pipelined(x2_sh, src_v, dst_v, (r0, r1, r2, r3),
                   (gs0, gs1, gs2, gs3), acc_sh, rpw)
    plsc.subcore_barrier()
    _par_copy(acc_sh, out_hbm.at[c], s)


_agg2_kernel = functools.partial(
    pl.kernel,
    out_type=jax.ShapeDtypeStruct((2, N_NODES, D_H2), jnp.float32),
    mesh=_sc_mesh,
    scratch_types=[
        pltpu.VMEM((EROWS // NW, EB), jnp.int32),
        pltpu.VMEM((EROWS // NW, EB), jnp.int32),
        pltpu.VMEM((EB, D_H2), jnp.float32),
        pltpu.VMEM((EB, D_H2), jnp.float32),
        pltpu.VMEM((EB, D_H2), jnp.float32),
        pltpu.VMEM((EB, D_H2), jnp.float32),
        pltpu.VMEM_SHARED((N_NODES, D_H2), jnp.float32),
        pltpu.VMEM_SHARED((N_NODES, D_H2), jnp.float32),
        pltpu.SemaphoreType.DMA,
        pltpu.SemaphoreType.DMA,
        pltpu.SemaphoreType.DMA,
        pltpu.SemaphoreType.DMA,
        pltpu.SemaphoreType.DMA,
    ],
    compiler_params=_sc_params,
)(_agg2_body)


def _z_body(p_ref, si_ref, b2_ref, z_ref):
    z_ref[...] = (p_ref[0] + p_ref[1]) * si_ref[...] + b2_ref[0][None, :]


def _z_combine(partials, s_in, b2):
    return pl.pallas_call(
        _z_body,
        out_shape=jax.ShapeDtypeStruct((N_NODES, D_H2), jnp.float32),
    )(partials, s_in, b2)


def _dec_body(zi_ref, zj_ref, out_ref):
    prod = lax.dot_general(zi_ref[...], zj_ref[...],
                           (((1,), (1,)), ((), ())),
                           preferred_element_type=jnp.float32)
    out_ref[...] = jax.nn.sigmoid(prod)


def _decoder(z):
    blk = 2048
    grid = pl.cdiv(N_NODES, blk)
    return pl.pallas_call(
        _dec_body,
        grid=(grid, grid),
        in_specs=[
            pl.BlockSpec((blk, D_H2), lambda i, j: (i, 0)),
            pl.BlockSpec((blk, D_H2), lambda i, j: (j, 0)),
        ],
        out_specs=pl.BlockSpec((blk, blk), lambda i, j: (i, j)),
        out_shape=jax.ShapeDtypeStruct((N_NODES, N_NODES), jnp.float32),
    )(z, z)


@jax.jit
def kernel(features, edge_index, W1, b1, W2, b2):
    src = edge_index[0]
    dst = edge_index[1]
    src2d = src.reshape(EROWS, EB)
    dst2d = dst.reshape(EROWS, EB)

    part_s, part_d = _deg_kernel(src, dst)
    s_out, s_in = _scales(part_s.reshape(NW, N_NODES), part_d.reshape(NW, N_NODES))

    x1 = _mm1(features, W1, s_out)
    zero1 = jnp.zeros((N_NODES, 128), jnp.float32)
    agg1 = _agg1_kernel(x1, src2d, dst2d, zero1)

    x2 = _mm2(agg1, s_in, s_out, b1.reshape(1, D_H1), W2)
    zero2 = jnp.zeros((N_NODES, D_H2), jnp.float32)
    p2 = _agg2_kernel(x2, src2d, dst2d, zero2)

    z = _z_combine(p2, s_in, b2.reshape(1, D_H2))
    return _decoder(z)
